# baseline (speedup 1.0000x reference)
import jax
import jax.numpy as jnp
from jax import lax
from jax.experimental import pallas as pl
from jax.experimental.pallas import tpu as pltpu
from jax.experimental.pallas import tpu_sc as plsc

N_ROWS = 320000
N_COLS = 128
N_SEG = 10000
SUB = 128
BLK = 512
NSUB = BLK // SUB
N_BLKS = N_ROWS // BLK
NC = 2
NS = 16
HALF = N_COLS // NC
NBUF = 2
ITERS = 40
OUT_ROWS_PER_TILE = N_SEG // NS


def kernel(x, batch):
    idx3 = batch.astype(jnp.int32).reshape(N_BLKS, NSUB, SUB)
    zeros_blk = jnp.zeros((OUT_ROWS_PER_TILE, HALF), jnp.float32)

    mesh = plsc.VectorSubcoreMesh(core_axis_name="c", subcore_axis_name="s")

    @pl.kernel(
        out_type=jax.ShapeDtypeStruct((N_SEG, N_COLS), jnp.float32),
        mesh=mesh,
        compiler_params=pltpu.CompilerParams(use_tc_tiling_on_sc=False),
        scratch_types=[
            pltpu.VMEM_SHARED((N_SEG, HALF), jnp.float32),
            pltpu.VMEM((NBUF, BLK, HALF), jnp.float32),
            pltpu.VMEM((NBUF, NSUB, SUB), jnp.int32),
            pltpu.SemaphoreType.DMA,
            pltpu.SemaphoreType.DMA,
            pltpu.SemaphoreType.DMA,
        ],
    )
    def seg_sum(x_hbm, idx_hbm, z_hbm, out_hbm, acc, rows_v, idx_v,
                sem0, sem1, sem_sc):
        c = lax.axis_index("c")
        s = lax.axis_index("s")
        sems = (sem0, sem1)

        def copies(i, b):
            cid = i * NS + s
            return (
                cid < N_BLKS,
                pltpu.make_async_copy(idx_hbm.at[cid], idx_v.at[b], sems[b]),
                pltpu.make_async_copy(
                    x_hbm.at[pl.ds(cid * BLK, BLK), pl.ds(c * HALF, HALF)],
                    rows_v.at[b], sems[b]),
            )

        def issue(i, b):
            live, cp_i, cp_r = copies(i, b)

            @pl.when(live)
            def _():
                cp_i.start()
                cp_r.start()

        def fire_scatters(i, b):
            live, cp_i, cp_r = copies(i, b)

            @pl.when(live)
            def _():
                cp_i.wait()
                cp_r.wait()
                for j in range(NSUB):
                    pltpu.async_copy(rows_v.at[b, pl.ds(j * SUB, SUB)],
                                     acc.at[idx_v.at[b, j]], sem_sc, add=True)

        def drain_scatters(i, b):
            cid = i * NS + s

            @pl.when((i >= 0) & (cid < N_BLKS))
            def _():
                for j in range(NSUB):
                    pltpu.make_async_copy(
                        rows_v.at[b, pl.ds(j * SUB, SUB)],
                        acc.at[idx_v.at[b, j]], sem_sc).wait()

        issue(0, 0)
        pltpu.sync_copy(
            z_hbm, acc.at[pl.ds(s * OUT_ROWS_PER_TILE, OUT_ROWS_PER_TILE)])
        plsc.subcore_barrier()

        @pl.loop(0, ITERS, step=NBUF)
        def _(i0):
            for b in range(NBUF):
                drain_scatters(i0 + b - 1, 1 - b)
                issue(i0 + b + 1, 1 - b)
                fire_scatters(i0 + b, b)

        drain_scatters(ITERS - 1, (ITERS - 1) % NBUF)
        plsc.subcore_barrier()
        pltpu.sync_copy(
            acc.at[pl.ds(s * OUT_ROWS_PER_TILE, OUT_ROWS_PER_TILE)],
            out_hbm.at[pl.ds(s * OUT_ROWS_PER_TILE, OUT_ROWS_PER_TILE),
                       pl.ds(c * HALF, HALF)])

    return seg_sum(x, idx3, zeros_blk)


# speedup vs baseline: 1.0110x; 1.0110x over previous
import jax
import jax.numpy as jnp
from jax import lax
from jax.experimental import pallas as pl
from jax.experimental.pallas import tpu as pltpu
from jax.experimental.pallas import tpu_sc as plsc

N_ROWS = 320000
N_COLS = 128
N_SEG = 10000
SUB = 128
BLK = 640
NSUB = BLK // SUB
N_BLKS = N_ROWS // BLK
NC = 2
NS = 16
HALF = N_COLS // NC
NBUF = 2
ITERS = 32
OUT_ROWS_PER_TILE = N_SEG // NS


def kernel(x, batch):
    idx3 = batch.astype(jnp.int32).reshape(N_BLKS, NSUB, SUB)
    zeros_blk = jnp.zeros((OUT_ROWS_PER_TILE, HALF), jnp.float32)

    mesh = plsc.VectorSubcoreMesh(core_axis_name="c", subcore_axis_name="s")

    @pl.kernel(
        out_type=jax.ShapeDtypeStruct((N_SEG, N_COLS), jnp.float32),
        mesh=mesh,
        compiler_params=pltpu.CompilerParams(use_tc_tiling_on_sc=False),
        scratch_types=[
            pltpu.VMEM_SHARED((N_SEG, HALF), jnp.float32),
            pltpu.VMEM((NBUF, BLK, HALF), jnp.float32),
            pltpu.VMEM((NBUF, NSUB, SUB), jnp.int32),
            pltpu.SemaphoreType.DMA,
            pltpu.SemaphoreType.DMA,
            pltpu.SemaphoreType.DMA,
        ],
    )
    def seg_sum(x_hbm, idx_hbm, z_hbm, out_hbm, acc, rows_v, idx_v,
                sem0, sem1, sem_sc):
        c = lax.axis_index("c")
        s = lax.axis_index("s")
        sems = (sem0, sem1)

        def copies(i, b):
            cid = i * NS + s
            return (
                cid < N_BLKS,
                pltpu.make_async_copy(idx_hbm.at[cid], idx_v.at[b], sems[b]),
                pltpu.make_async_copy(
                    x_hbm.at[pl.ds(cid * BLK, BLK), pl.ds(c * HALF, HALF)],
                    rows_v.at[b], sems[b]),
            )

        def issue(i, b):
            live, cp_i, cp_r = copies(i, b)

            @pl.when(live)
            def _():
                cp_i.start()
                cp_r.start()

        def fire_scatters(i, b):
            live, cp_i, cp_r = copies(i, b)

            @pl.when(live)
            def _():
                cp_i.wait()
                cp_r.wait()
                for j in range(NSUB):
                    pltpu.async_copy(rows_v.at[b, pl.ds(j * SUB, SUB)],
                                     acc.at[idx_v.at[b, j]], sem_sc, add=True)

        def drain_scatters(i, b):
            cid = i * NS + s

            @pl.when((i >= 0) & (cid < N_BLKS))
            def _():
                for j in range(NSUB):
                    pltpu.make_async_copy(
                        rows_v.at[b, pl.ds(j * SUB, SUB)],
                        acc.at[idx_v.at[b, j]], sem_sc).wait()

        issue(0, 0)
        pltpu.sync_copy(
            z_hbm, acc.at[pl.ds(s * OUT_ROWS_PER_TILE, OUT_ROWS_PER_TILE)])
        plsc.subcore_barrier()

        @pl.loop(0, ITERS, step=NBUF)
        def _(i0):
            for b in range(NBUF):
                drain_scatters(i0 + b - 1, 1 - b)
                issue(i0 + b + 1, 1 - b)
                fire_scatters(i0 + b, b)

        drain_scatters(ITERS - 1, (ITERS - 1) % NBUF)
        plsc.subcore_barrier()
        pltpu.sync_copy(
            acc.at[pl.ds(s * OUT_ROWS_PER_TILE, OUT_ROWS_PER_TILE)],
            out_hbm.at[pl.ds(s * OUT_ROWS_PER_TILE, OUT_ROWS_PER_TILE),
                       pl.ds(c * HALF, HALF)])

    return seg_sum(x, idx3, zeros_blk)
